# Initial kernel scaffold; baseline (speedup 1.0000x reference)
import functools

import jax
import jax.numpy as jnp
from jax import lax
from jax.experimental import pallas as pl
from jax.experimental.pallas import tpu as pltpu
from jax.experimental.pallas import tpu_sc as plsc

_NC = 2
_NS = 16
_NW = _NC * _NS
_K = 128
_DEGW = 16


def _sc_aggregate(N, C, EPAD):
    EPT = EPAD // _NW
    T = EPT // _K
    NPAD = -(-(N + 1) // (_NS * _K)) * (_NS * _K)
    ZPT = NPAD // _NS // _K
    RPT = N // _NS

    mesh = plsc.VectorSubcoreMesh(core_axis_name="c", subcore_axis_name="s")

    @functools.partial(
        pl.kernel,
        out_type=(jax.ShapeDtypeStruct((_NC, N, C), jnp.float32),
                  jax.ShapeDtypeStruct((_NC, N, _DEGW), jnp.float32)),
        mesh=mesh,
        scratch_types=[
            pltpu.VMEM((_K,), jnp.int32),
            pltpu.VMEM((_K,), jnp.int32),
            pltpu.VMEM((_K, C), jnp.float32),
            pltpu.VMEM((_K, _DEGW), jnp.float32),
            pltpu.VMEM((_K, _DEGW), jnp.float32),
            pltpu.VMEM_SHARED((None, C), jnp.float32) if False else pltpu.VMEM_SHARED((-(-(10000 + 1) // (16 * 128)) * (16 * 128), C), jnp.float32),
        ],
    )
    def agg(*args):
        pass

    return agg


# speedup vs baseline: 3.1175x; 3.1175x over previous
import functools

import jax
import jax.numpy as jnp
from jax import lax
from jax.experimental import pallas as pl
from jax.experimental.pallas import tpu as pltpu
from jax.experimental.pallas import tpu_sc as plsc

_NC = 2
_NS = 16
_NW = _NC * _NS
_K = 128
_DEGW = 16


def _sc_aggregate(N, C, EPAD):
    CH = C // 2
    EPT = EPAD // _NS
    T = EPT // _K
    NPAD = -(-(N + 1) // _K) * _K
    TZ = NPAD // _K
    ZPT = -(-TZ // _NS)
    RPT = NPAD // _NS
    assert RPT % 8 == 0 and NPAD % _NS == 0 and EPT % _K == 0

    mesh = plsc.VectorSubcoreMesh(core_axis_name="c", subcore_axis_name="s",
                                  num_cores=_NC, num_subcores=_NS)

    @functools.partial(
        pl.kernel,
        out_type=(jax.ShapeDtypeStruct((_NC, NPAD, CH), jnp.float32),
                  jax.ShapeDtypeStruct((NPAD, _DEGW), jnp.float32)),
        mesh=mesh,
        scratch_types=[
            pltpu.VMEM((_K,), jnp.int32),
            pltpu.VMEM((_K,), jnp.int32),
            pltpu.VMEM((_K, CH), jnp.float32),
            pltpu.VMEM((_K, _DEGW), jnp.float32),
            pltpu.VMEM((_K, _DEGW), jnp.float32),
            pltpu.VMEM_SHARED((NPAD, CH), jnp.float32),
            pltpu.VMEM_SHARED((NPAD, _DEGW), jnp.float32),
            pltpu.SemaphoreType.DMA,
        ],
        compiler_params=pltpu.CompilerParams(use_tc_tiling_on_sc=False),
    )
    def agg(xlo_hbm, xhi_hbm, cols_hbm, rows_hbm, sum_hbm, deg_hbm,
            idxc, idxr, rows_v, ones_v, zeros16_v, acc_sh, deg_sh, sem):
        c = lax.axis_index("c")
        s = lax.axis_index("s")

        zeros = jnp.zeros((16,), jnp.float32)
        ones = jnp.ones((16,), jnp.float32)

        def memset_row(i, carry):
            for j in range(CH // 16):
                rows_v[i, pl.ds(16 * j, 16)] = zeros
            ones_v[i, pl.ds(0, 16)] = ones
            zeros16_v[i, pl.ds(0, 16)] = zeros
            return carry
        lax.fori_loop(0, _K, memset_row, 0)

        def zero_chunk(k, carry):
            g = k * _NS + s
            @pl.when(g < TZ)
            def _():
                r0 = g * _K
                pltpu.sync_copy(rows_v, acc_sh.at[pl.ds(r0, _K)])
                pltpu.sync_copy(zeros16_v, deg_sh.at[pl.ds(r0, _K)])
            return carry
        lax.fori_loop(0, ZPT, zero_chunk, 0)

        plsc.subcore_barrier()

        ebase = s * EPT

        def edge_chunk(t, carry):
            e0 = ebase + t * _K
            pltpu.sync_copy(cols_hbm.at[pl.ds(e0, _K)], idxc)
            pltpu.sync_copy(rows_hbm.at[pl.ds(e0, _K)], idxr)

            @pl.when(c == 0)
            def _():
                pltpu.async_copy(xlo_hbm.at[idxc], rows_v, sem).wait()

            @pl.when(c == 1)
            def _():
                pltpu.async_copy(xhi_hbm.at[idxc], rows_v, sem).wait()

            pltpu.sync_copy(rows_v, acc_sh.at[idxr], add=True)

            @pl.when(c == 0)
            def _():
                pltpu.sync_copy(ones_v, deg_sh.at[idxr], add=True)
            return carry
        lax.fori_loop(0, T, edge_chunk, 0)

        plsc.subcore_barrier()

        r0 = s * RPT
        pltpu.sync_copy(acc_sh.at[pl.ds(r0, RPT)], sum_hbm.at[c, pl.ds(r0, RPT)])

        @pl.when(c == 0)
        def _():
            pltpu.sync_copy(deg_sh.at[pl.ds(r0, RPT)], deg_hbm.at[pl.ds(r0, RPT)])

    return agg


def _dense_layer(N, C, H, relu, BN=2000):
    CH = C // 2

    def body(x_ref, plo_ref, phi_ref, deg_ref, w_ref, b_ref, o_ref):
        inv = 1.0 / jnp.maximum(deg_ref[:, 0:1], 1.0)
        wx = w_ref[:, :C]
        wa_lo = w_ref[:, C:C + CH]
        wa_hi = w_ref[:, C + CH:]
        y = lax.dot_general(x_ref[...], wx, (((1,), (1,)), ((), ())),
                            preferred_element_type=jnp.float32)
        y = y + lax.dot_general(plo_ref[...] * inv, wa_lo,
                                (((1,), (1,)), ((), ())),
                                preferred_element_type=jnp.float32)
        y = y + lax.dot_general(phi_ref[...] * inv, wa_hi,
                                (((1,), (1,)), ((), ())),
                                preferred_element_type=jnp.float32)
        y = y + b_ref[...]
        o_ref[...] = jnp.maximum(y, 0.0) if relu else y

    return pl.pallas_call(
        body,
        grid=(N // BN,),
        in_specs=[
            pl.BlockSpec((BN, C), lambda i: (i, 0)),
            pl.BlockSpec((BN, CH), lambda i: (i, 0)),
            pl.BlockSpec((BN, CH), lambda i: (i, 0)),
            pl.BlockSpec((BN, _DEGW), lambda i: (i, 0)),
            pl.BlockSpec((H, 2 * C), lambda i: (0, 0)),
            pl.BlockSpec((1, H), lambda i: (0, 0)),
        ],
        out_specs=pl.BlockSpec((BN, H), lambda i: (i, 0)),
        out_shape=jax.ShapeDtypeStruct((N, H), jnp.float32),
    )


def kernel(x, edge_index, W1, b1, W2, b2):
    N, C = x.shape
    H = W1.shape[0]
    O = W2.shape[0]
    E = edge_index.shape[1]
    EPAD = -(-E // (_NW * _K)) * (_NW * _K)

    rows = edge_index[0]
    cols = edge_index[1]
    pad = EPAD - E
    if pad:
        rows = jnp.concatenate([rows, jnp.full((pad,), N, jnp.int32)])
        cols = jnp.concatenate([cols, jnp.zeros((pad,), jnp.int32)])

    sums, deg = _sc_aggregate(N, C, EPAD)(
        x[:, :C // 2], x[:, C // 2:], cols, rows)
    h = _dense_layer(N, C, H, True)(
        x, sums[0, :N], sums[1, :N], deg[:N], W1, b1.reshape(1, H))

    sums2, deg2 = _sc_aggregate(N, H, EPAD)(
        h[:, :H // 2], h[:, H // 2:], cols, rows)
    out = _dense_layer(N, H, O, False)(
        h, sums2[0, :N], sums2[1, :N], deg2[:N], W2, b2.reshape(1, O))
    return out


# speedup vs baseline: 5.8220x; 1.8675x over previous
import functools

import jax
import jax.numpy as jnp
from jax import lax
from jax.experimental import pallas as pl
from jax.experimental.pallas import tpu as pltpu
from jax.experimental.pallas import tpu_sc as plsc

_NC = 2
_NS = 16
_NW = _NC * _NS
_K = 128
_DEGW = 16


def _sc_aggregate(N, C, EPAD):
    CH = C // 2
    EPT = EPAD // _NS
    T = EPT // _K
    NPAD = -(-(N + 1) // _K) * _K
    TZ = NPAD // _K
    ZPT = -(-TZ // _NS)
    RPT = NPAD // _NS
    assert RPT % 8 == 0 and NPAD % _NS == 0 and EPT % _K == 0

    mesh = plsc.VectorSubcoreMesh(core_axis_name="c", subcore_axis_name="s",
                                  num_cores=_NC, num_subcores=_NS)

    assert T % 2 == 0

    @functools.partial(
        pl.kernel,
        out_type=(jax.ShapeDtypeStruct((_NC, NPAD, CH), jnp.float32),
                  jax.ShapeDtypeStruct((NPAD, _DEGW), jnp.float32)),
        mesh=mesh,
        scratch_types=[
            pltpu.VMEM((T, _K), jnp.int32),
            pltpu.VMEM((T, _K), jnp.int32),
            pltpu.VMEM((_K, CH), jnp.float32),
            pltpu.VMEM((_K, CH), jnp.float32),
            pltpu.VMEM((_K, _DEGW), jnp.float32),
            pltpu.VMEM((_K, _DEGW), jnp.float32),
            pltpu.VMEM_SHARED((NPAD, CH), jnp.float32),
            pltpu.VMEM_SHARED((NPAD, _DEGW), jnp.float32),
            pltpu.SemaphoreType.DMA,
            pltpu.SemaphoreType.DMA,
            pltpu.SemaphoreType.DMA,
            pltpu.SemaphoreType.DMA,
            pltpu.SemaphoreType.DMA,
            pltpu.SemaphoreType.DMA,
        ],
        compiler_params=pltpu.CompilerParams(use_tc_tiling_on_sc=False),
    )
    def agg(xs_hbm, cols_hbm, rows_hbm, sum_hbm, deg_hbm,
            idxc_all, idxr_all, rows0, rows1, ones_v, zeros16_v,
            acc_sh, deg_sh, gsem0, gsem1, ssem0, ssem1, dsem0, dsem1):
        c = lax.axis_index("c")
        s = lax.axis_index("s")

        zeros = jnp.zeros((16,), jnp.float32)
        ones = jnp.ones((16,), jnp.float32)

        def memset_row(i, carry):
            for j in range(CH // 16):
                rows0[i, pl.ds(16 * j, 16)] = zeros
            ones_v[i, pl.ds(0, 16)] = ones
            zeros16_v[i, pl.ds(0, 16)] = zeros
            return carry
        lax.fori_loop(0, _K, memset_row, 0)

        def zero_chunk(k, carry):
            g = k * _NS + s
            @pl.when(g < TZ)
            def _():
                r0 = g * _K
                pltpu.sync_copy(rows0, acc_sh.at[pl.ds(r0, _K)])
                pltpu.sync_copy(zeros16_v, deg_sh.at[pl.ds(r0, _K)])
            return carry
        lax.fori_loop(0, ZPT, zero_chunk, 0)

        pltpu.sync_copy(cols_hbm.at[pl.ds(s * T, T)], idxc_all)
        pltpu.sync_copy(rows_hbm.at[pl.ds(s * T, T)], idxr_all)

        plsc.subcore_barrier()

        bufs = ((rows0, gsem0, ssem0, dsem0), (rows1, gsem1, ssem1, dsem1))
        xc = xs_hbm.at[c]

        def gather(g, b):
            return pltpu.async_copy(xc.at[idxc_all.at[g]], bufs[b][0], bufs[b][1])

        def gather_wait(g, b):
            pltpu.make_async_copy(xc.at[idxc_all.at[g]], bufs[b][0], bufs[b][1]).wait()

        def scatter(g, b):
            pltpu.async_copy(bufs[b][0], acc_sh.at[idxr_all.at[g]], bufs[b][2], add=True)
            pltpu.async_copy(ones_v, deg_sh.at[idxr_all.at[g]], bufs[b][3], add=True)

        def scatter_wait(g, b):
            pltpu.make_async_copy(bufs[b][0], acc_sh.at[idxr_all.at[g]], bufs[b][2]).wait()
            pltpu.make_async_copy(ones_v, deg_sh.at[idxr_all.at[g]], bufs[b][3]).wait()

        def pipe(u, carry):
            for b in range(2):
                g = u * 2 + b

                @pl.when(g >= 2)
                def _():
                    scatter_wait(g - 2, b)

                gather(g, b)

                @pl.when(g >= 1)
                def _():
                    gather_wait(g - 1, 1 - b)
                    scatter(g - 1, 1 - b)
            return carry
        lax.fori_loop(0, T // 2, pipe, 0)

        gather_wait(T - 1, 1)
        scatter(T - 1, 1)
        scatter_wait(T - 2, 0)
        scatter_wait(T - 1, 1)

        plsc.subcore_barrier()

        r0 = s * RPT
        pltpu.sync_copy(acc_sh.at[pl.ds(r0, RPT)], sum_hbm.at[c, pl.ds(r0, RPT)])

        @pl.when(c == 0)
        def _():
            pltpu.sync_copy(deg_sh.at[pl.ds(r0, RPT)], deg_hbm.at[pl.ds(r0, RPT)])

    return agg


def _dense_layer(N, C, H, relu, BN=2000):
    CH = C // 2

    def body(x_ref, plo_ref, phi_ref, deg_ref, w_ref, b_ref, o_ref):
        inv = 1.0 / jnp.maximum(deg_ref[:, 0:1], 1.0)
        wx = w_ref[:, :C]
        wa_lo = w_ref[:, C:C + CH]
        wa_hi = w_ref[:, C + CH:]
        y = lax.dot_general(x_ref[...], wx, (((1,), (1,)), ((), ())),
                            preferred_element_type=jnp.float32)
        y = y + lax.dot_general(plo_ref[...] * inv, wa_lo,
                                (((1,), (1,)), ((), ())),
                                preferred_element_type=jnp.float32)
        y = y + lax.dot_general(phi_ref[...] * inv, wa_hi,
                                (((1,), (1,)), ((), ())),
                                preferred_element_type=jnp.float32)
        y = y + b_ref[...]
        o_ref[...] = jnp.maximum(y, 0.0) if relu else y

    return pl.pallas_call(
        body,
        grid=(N // BN,),
        in_specs=[
            pl.BlockSpec((BN, C), lambda i: (i, 0)),
            pl.BlockSpec((BN, CH), lambda i: (i, 0)),
            pl.BlockSpec((BN, CH), lambda i: (i, 0)),
            pl.BlockSpec((BN, _DEGW), lambda i: (i, 0)),
            pl.BlockSpec((H, 2 * C), lambda i: (0, 0)),
            pl.BlockSpec((1, H), lambda i: (0, 0)),
        ],
        out_specs=pl.BlockSpec((BN, H), lambda i: (i, 0)),
        out_shape=jax.ShapeDtypeStruct((N, H), jnp.float32),
    )


def kernel(x, edge_index, W1, b1, W2, b2):
    N, C = x.shape
    H = W1.shape[0]
    O = W2.shape[0]
    E = edge_index.shape[1]
    EPAD = -(-E // (_NW * _K)) * (_NW * _K)

    rows = edge_index[0]
    cols = edge_index[1]
    pad = EPAD - E
    if pad:
        rows = jnp.concatenate([rows, jnp.full((pad,), N, jnp.int32)])
        cols = jnp.concatenate([cols, jnp.zeros((pad,), jnp.int32)])
    rows = rows.reshape(EPAD // _K, _K)
    cols = cols.reshape(EPAD // _K, _K)

    xs = jnp.stack([x[:, :C // 2], x[:, C // 2:]])
    sums, deg = _sc_aggregate(N, C, EPAD)(xs, cols, rows)
    h = _dense_layer(N, C, H, True)(
        x, sums[0, :N], sums[1, :N], deg[:N], W1, b1.reshape(1, H))

    hs = jnp.stack([h[:, :H // 2], h[:, H // 2:]])
    sums2, deg2 = _sc_aggregate(N, H, EPAD)(hs, cols, rows)
    out = _dense_layer(N, H, O, False)(
        h, sums2[0, :N], sums2[1, :N], deg2[:N], W2, b2.reshape(1, O))
    return out
